# baseline (speedup 1.0000x reference)
import functools

import jax
import jax.numpy as jnp
from jax import lax
from jax.experimental import pallas as pl
from jax.experimental.pallas import tpu as pltpu
from jax.experimental.pallas import tpu_sc as plsc

_B = 16384
_F = 32
_N = 1000000
_NW = 32
_BPW = _B // _NW
_NCHUNK = 4
_CHUNK = _BPW // _NCHUNK
_PACK = 4
_ROWW = _PACK * _F

_PB = 8000
_CB = _PB * _PACK
_PGRID = -(-_N // _CB)
_NPROWS = _PB * _PGRID


def _pack_body(x_ref, o_ref):
    y = x_ref[...].T
    o_ref[...] = jnp.concatenate(
        [y[k * _PB:(k + 1) * _PB] for k in range(_PACK)], axis=1)


_tc_pack = pl.pallas_call(
    _pack_body,
    grid=(_PGRID,),
    in_specs=[pl.BlockSpec((_F, _CB), lambda c: (0, c))],
    out_specs=pl.BlockSpec((_PB, _ROWW), lambda c: (c, 0)),
    out_shape=jax.ShapeDtypeStruct((_NPROWS, _ROWW), jnp.float32),
)


def _gather_u_body(up_ref, uemb_ref, out_ref, upix, ubuf, sem):
    nc = 2
    wid = lax.axis_index("s") * nc + lax.axis_index("c")
    pltpu.sync_copy(up_ref.at[wid], upix)
    copies = [pltpu.async_copy(uemb_ref.at[upix.at[j]],
                               ubuf.at[pl.ds(j * _CHUNK, _CHUNK)], sem)
              for j in range(_NCHUNK)]
    for cp in copies:
        cp.wait()
    pltpu.sync_copy(ubuf, out_ref.at[pl.ds(wid * _BPW, _BPW)])


@functools.partial(
    pl.kernel,
    mesh=plsc.VectorSubcoreMesh(core_axis_name="c", subcore_axis_name="s"),
    out_type=jax.ShapeDtypeStruct((_B, _ROWW), jnp.float32),
    compiler_params=pltpu.CompilerParams(
        needs_layout_passes=False, use_tc_tiling_on_sc=True),
    scratch_types=[
        pltpu.VMEM((_NCHUNK, _CHUNK), jnp.int32),
        pltpu.VMEM((_BPW, _ROWW), jnp.float32),
        pltpu.SemaphoreType.DMA,
    ],
)
def _sc_gather_u(up_ref, uemb_ref, out_ref, upix, ubuf, sem):
    _gather_u_body(up_ref, uemb_ref, out_ref, upix, ubuf, sem)


def _norm_body(ud_ref, item_ref, uraw_ref, iemb_ref, w_ref, b_ref, out_ref,
               udiv, iidx, uvm, irows, wv, bv, outv, sem):
    nc = 2
    wid = lax.axis_index("s") * nc + lax.axis_index("c")

    pltpu.sync_copy(ud_ref.at[wid], udiv)
    pltpu.sync_copy(item_ref.at[wid], iidx)
    pltpu.sync_copy(uraw_ref.at[pl.ds(wid * _BPW, _BPW)], uvm)
    pltpu.sync_copy(w_ref, wv)
    pltpu.sync_copy(b_ref, bv)

    copies = [pltpu.async_copy(iemb_ref.at[iidx.at[j]],
                               irows.at[pl.ds(j * _CHUNK, _CHUNK)], sem)
              for j in range(_NCHUNK)]
    for cp in copies:
        cp.wait()

    iot = lax.iota(jnp.int32, 16)
    w_vec = wv[...]
    b_vec = bv[...]

    def group(g, carry):
        rows = g * 16 + iot
        du = udiv[pl.ds(g * 16, 16)]
        acc = jnp.zeros((16,), jnp.float32)
        for f in range(_F):
            fcol = jnp.full((16,), f, jnp.int32)
            u = plsc.load_gather(uvm, [rows, du * _F + f])
            i = plsc.load_gather(irows, [rows, fcol])
            d = u - i
            acc = acc + d * d
        half = acc * 0.5
        bits = plsc.bitcast(acc, jnp.int32)
        bits = jnp.int32(0x5F3759DF) - (bits >> 1)
        y = plsc.bitcast(bits, jnp.float32)
        for _ in range(3):
            y = y * (1.5 - half * y * y)
        p2 = acc * y
        outv[pl.ds(g * 16, 16)] = p2 * w_vec + b_vec
        return carry

    lax.fori_loop(0, _BPW // 16, group, 0)
    pltpu.sync_copy(outv, out_ref.at[pl.ds(wid * _BPW, _BPW)])


@functools.partial(
    pl.kernel,
    mesh=plsc.VectorSubcoreMesh(core_axis_name="c", subcore_axis_name="s"),
    out_type=jax.ShapeDtypeStruct((_B,), jnp.float32),
    compiler_params=pltpu.CompilerParams(
        needs_layout_passes=False, use_tc_tiling_on_sc=False),
    scratch_types=[
        pltpu.VMEM((_BPW,), jnp.int32),
        pltpu.VMEM((_NCHUNK, _CHUNK), jnp.int32),
        pltpu.VMEM((_BPW, _ROWW), jnp.float32),
        pltpu.VMEM((_BPW, _F), jnp.float32),
        pltpu.VMEM((16,), jnp.float32),
        pltpu.VMEM((16,), jnp.float32),
        pltpu.VMEM((_BPW,), jnp.float32),
        pltpu.SemaphoreType.DMA,
    ],
)
def _sc_norm(ud_ref, item_ref, uraw_ref, iemb_ref, w_ref, b_ref, out_ref,
             udiv, iidx, uvm, irows, wv, bv, outv, sem):
    _norm_body(ud_ref, item_ref, uraw_ref, iemb_ref, w_ref, b_ref, out_ref,
               udiv, iidx, uvm, irows, wv, bv, outv, sem)


def kernel(user, item, user_emb, item_emb, lin_w, lin_b):
    user = user.astype(jnp.int32)
    item = item.astype(jnp.int32)
    up = ((user // _CB) * _PB + user % _PB).reshape(_NW, _NCHUNK, _CHUNK)
    ud = ((user % _CB) // _PB).reshape(_NW, _BPW)
    item_r = item.reshape(_NW, _NCHUNK, _CHUNK)
    w16 = jnp.full((16,), lin_w.reshape(()), jnp.float32)
    b16 = jnp.full((16,), lin_b.reshape(()), jnp.float32)
    packed_u = _tc_pack(user_emb.T)
    uraw = _sc_gather_u(up, packed_u)
    return _sc_norm(ud, item_r, uraw, item_emb, w16, b16)


# speedup vs baseline: 1.1708x; 1.1708x over previous
import functools

import jax
import jax.numpy as jnp
from jax import lax
from jax.experimental import pallas as pl
from jax.experimental.pallas import tpu as pltpu
from jax.experimental.pallas import tpu_sc as plsc

_B = 16384
_F = 32
_N = 1000000
_NW = 32
_BPW = _B // _NW
_NCHUNK = 4
_CHUNK = _BPW // _NCHUNK
_PACK = 4
_NPACK = _N // _PACK
_GPC = _CHUNK // 16

_PB = 8000
_CB = _PB * _PACK
_PGRID = -(-_N // _CB)
_NPROWS = _PB * _PGRID


def _pack_body(x_ref, o_ref):
    y = x_ref[...].T
    o_ref[...] = jnp.concatenate(
        [y[k * _PB:(k + 1) * _PB] for k in range(_PACK)], axis=1)


_tc_pack = pl.pallas_call(
    _pack_body,
    grid=(_PGRID,),
    in_specs=[pl.BlockSpec((_F, _CB), lambda c: (0, c))],
    out_specs=pl.BlockSpec((_PB, _PACK * _F), lambda c: (c, 0)),
    out_shape=jax.ShapeDtypeStruct((_NPROWS, _PACK * _F), jnp.float32),
)


def _sc_body(up_ref, ud_ref, ip_ref, id_ref, uemb_ref, iemb_ref, wb_ref,
             out_ref, upix, udiv, ipix, idiv, ubuf, ibuf, wbv, outv, sem):
    nc = 2
    wid = lax.axis_index("s") * nc + lax.axis_index("c")

    pltpu.sync_copy(up_ref.at[wid], upix)
    pltpu.sync_copy(ud_ref.at[wid], udiv)
    pltpu.sync_copy(ip_ref.at[wid], ipix)
    pltpu.sync_copy(id_ref.at[wid], idiv)
    pltpu.sync_copy(wb_ref, wbv)

    def fire(j):
        slot = j % 2
        return (
            pltpu.async_copy(uemb_ref.at[upix.at[j]], ubuf.at[slot], sem),
            pltpu.async_copy(iemb_ref.at[ipix.at[j]], ibuf.at[slot], sem),
        )

    iot = lax.iota(jnp.int32, 16)
    w_vec = wbv[pl.ds(0, 16)]
    b_vec = wbv[pl.ds(16, 16)]

    pending = fire(0)
    for j in range(_NCHUNK):
        nxt = fire(j + 1) if j + 1 < _NCHUNK else None
        for cp in pending:
            cp.wait()
        slot = j % 2

        def group(g, carry, j=j, slot=slot):
            rows = g * 16 + iot
            du = udiv[j, pl.ds(g * 16, 16)]
            di = idiv[j, pl.ds(g * 16, 16)]
            acc = jnp.zeros((16,), jnp.float32)
            for f in range(_F):
                u = plsc.load_gather(ubuf, [jnp.full((16,), slot, jnp.int32),
                                            rows, du * _F + f])
                i = plsc.load_gather(ibuf, [jnp.full((16,), slot, jnp.int32),
                                            rows, di * _F + f])
                d = u - i
                acc = acc + d * d
            half = acc * 0.5
            bits = plsc.bitcast(acc, jnp.int32)
            bits = jnp.int32(0x5F3759DF) - (bits >> 1)
            y = plsc.bitcast(bits, jnp.float32)
            for _ in range(3):
                y = y * (1.5 - half * y * y)
            p2 = acc * y
            outv[pl.ds(j * _CHUNK + g * 16, 16)] = p2 * w_vec + b_vec
            return carry

        lax.fori_loop(0, _GPC, group, 0)
        pending = nxt

    pltpu.sync_copy(outv, out_ref.at[pl.ds(wid * _BPW, _BPW)])


@functools.partial(
    pl.kernel,
    mesh=plsc.VectorSubcoreMesh(core_axis_name="c", subcore_axis_name="s"),
    out_type=jax.ShapeDtypeStruct((_B,), jnp.float32),
    compiler_params=pltpu.CompilerParams(
        needs_layout_passes=False, use_tc_tiling_on_sc=True),
    scratch_types=[
        pltpu.VMEM((_NCHUNK, _CHUNK), jnp.int32),
        pltpu.VMEM((_NCHUNK, _CHUNK), jnp.int32),
        pltpu.VMEM((_NCHUNK, _CHUNK), jnp.int32),
        pltpu.VMEM((_NCHUNK, _CHUNK), jnp.int32),
        pltpu.VMEM((2, _CHUNK, _PACK * _F), jnp.float32),
        pltpu.VMEM((2, _CHUNK, _PACK * _F), jnp.float32),
        pltpu.VMEM((32,), jnp.float32),
        pltpu.VMEM((_BPW,), jnp.float32),
        pltpu.SemaphoreType.DMA,
    ],
)
def _sc_rating(up_ref, ud_ref, ip_ref, id_ref, uemb_ref, iemb_ref, wb_ref,
               out_ref, upix, udiv, ipix, idiv, ubuf, ibuf, wbv, outv, sem):
    _sc_body(up_ref, ud_ref, ip_ref, id_ref, uemb_ref, iemb_ref, wb_ref,
             out_ref, upix, udiv, ipix, idiv, ubuf, ibuf, wbv, outv, sem)


def kernel(user, item, user_emb, item_emb, lin_w, lin_b):
    user = user.astype(jnp.int32)
    item = item.astype(jnp.int32)
    up = ((user // _CB) * _PB + user % _PB).reshape(_NW, _NCHUNK, _CHUNK)
    ud = ((user % _CB) // _PB).reshape(_NW, _NCHUNK, _CHUNK)
    ip = ((item // _CB) * _PB + item % _PB).reshape(_NW, _NCHUNK, _CHUNK)
    idv = ((item % _CB) // _PB).reshape(_NW, _NCHUNK, _CHUNK)
    wb = jnp.concatenate([jnp.full((16,), lin_w.reshape(()), jnp.float32),
                          jnp.full((16,), lin_b.reshape(()), jnp.float32)])
    packed_u = _tc_pack(user_emb.T)
    packed_i = _tc_pack(item_emb.T)
    return _sc_rating(up, ud, ip, idv, packed_u, packed_i, wb)
